# baseline (device time: 24268 ns/iter reference)
import jax
import jax.numpy as jnp
from jax import lax
from jax.experimental import pallas as pl
from jax.experimental.pallas import tpu as pltpu

M = 2048
N = 1024
HALF_M = M // 2
HALF_N = N // 2
C = 8
CHUNK = HALF_M // C


def kernel(x):
    def body(x_ref, out_ref, xstage, ysend, yrecv, load_sems,
             ysend_sems, yrecv_sems, xsend_sems, xrecv_sems):
        my_x = lax.axis_index("x")
        my_y = lax.axis_index("y")

        row0 = my_x * HALF_M

        loads = []
        for k in range(C):
            cp = pltpu.make_async_copy(
                x_ref.at[0, pl.ds(row0 + k * CHUNK, CHUNK), :],
                xstage.at[pl.ds(k * CHUNK, CHUNK), :],
                load_sems.at[k],
            )
            cp.start()
            loads.append(cp)

        barrier = pltpu.get_barrier_semaphore()
        pl.semaphore_signal(
            barrier, inc=1, device_id=(my_x, 1 - my_y),
            device_id_type=pl.DeviceIdType.MESH,
        )
        pl.semaphore_signal(
            barrier, inc=1, device_id=(1 - my_x, my_y),
            device_id_type=pl.DeviceIdType.MESH,
        )
        pl.semaphore_wait(barrier, 2)

        y_rdmas = []
        for k in range(C):
            loads[k].wait()

            @pl.when(my_y == 0)
            def _(k=k):
                ysend[pl.ds(k * CHUNK, CHUNK), :] = xstage[
                    pl.ds(k * CHUNK, CHUNK), HALF_N:
                ].astype(jnp.bfloat16)

            @pl.when(my_y == 1)
            def _(k=k):
                ysend[pl.ds(k * CHUNK, CHUNK), :] = xstage[
                    pl.ds(k * CHUNK, CHUNK), :HALF_N
                ].astype(jnp.bfloat16)

            rdma = pltpu.make_async_remote_copy(
                src_ref=ysend.at[pl.ds(k * CHUNK, CHUNK), :],
                dst_ref=yrecv.at[pl.ds(k * CHUNK, CHUNK), :],
                send_sem=ysend_sems.at[k],
                recv_sem=yrecv_sems.at[k],
                device_id=(my_x, 1 - my_y),
                device_id_type=pl.DeviceIdType.MESH,
            )
            rdma.start()
            y_rdmas.append(rdma)

        x_rdmas = []
        for k in range(C):
            y_rdmas[k].wait_recv()

            @pl.when(my_y == 0)
            def _(k=k):
                out_ref[pl.ds(row0 + k * CHUNK, CHUNK), :] = (
                    xstage[pl.ds(k * CHUNK, CHUNK), :HALF_N].astype(jnp.bfloat16)
                    + yrecv[pl.ds(k * CHUNK, CHUNK), :]
                )

            @pl.when(my_y == 1)
            def _(k=k):
                out_ref[pl.ds(row0 + k * CHUNK, CHUNK), :] = (
                    xstage[pl.ds(k * CHUNK, CHUNK), HALF_N:].astype(jnp.bfloat16)
                    + yrecv[pl.ds(k * CHUNK, CHUNK), :]
                )

            rdma = pltpu.make_async_remote_copy(
                src_ref=out_ref.at[pl.ds(row0 + k * CHUNK, CHUNK), :],
                dst_ref=out_ref.at[pl.ds(row0 + k * CHUNK, CHUNK), :],
                send_sem=xsend_sems.at[k],
                recv_sem=xrecv_sems.at[k],
                device_id=(1 - my_x, my_y),
                device_id_type=pl.DeviceIdType.MESH,
            )
            rdma.start()
            x_rdmas.append(rdma)

        for k in range(C):
            y_rdmas[k].wait_send()
            x_rdmas[k].wait()

    return pl.pallas_call(
        body,
        out_shape=jax.ShapeDtypeStruct((M, HALF_N), jnp.bfloat16),
        in_specs=[pl.BlockSpec(memory_space=pltpu.MemorySpace.HBM)],
        out_specs=pl.BlockSpec(memory_space=pltpu.MemorySpace.VMEM),
        scratch_shapes=[
            pltpu.VMEM((HALF_M, N), jnp.float32),
            pltpu.VMEM((HALF_M, HALF_N), jnp.bfloat16),
            pltpu.VMEM((HALF_M, HALF_N), jnp.bfloat16),
            pltpu.SemaphoreType.DMA((C,)),
            pltpu.SemaphoreType.DMA((C,)),
            pltpu.SemaphoreType.DMA((C,)),
            pltpu.SemaphoreType.DMA((C,)),
            pltpu.SemaphoreType.DMA((C,)),
        ],
        compiler_params=pltpu.CompilerParams(
            collective_id=0,
            vmem_limit_bytes=100 * 1024 * 1024,
        ),
    )(x)


# device time: 22929 ns/iter; 1.0584x vs baseline; 1.0584x over previous
import jax
import jax.numpy as jnp
from jax import lax
from jax.experimental import pallas as pl
from jax.experimental.pallas import tpu as pltpu

M = 2048
N = 1024
HALF_M = M // 2
HALF_N = N // 2
C = 16
CHUNK = HALF_M // C


def kernel(x):
    def body(x_ref, out_ref, xstage, ysend, yrecv, oacc, peer_sems, mine_sems,
             ysend_sems, yrecv_sems, xsend_sems, xrecv_sems, store_sems):
        my_x = lax.axis_index("x")
        my_y = lax.axis_index("y")

        row0 = my_x * HALF_M
        col_me = my_y * HALF_N
        col_peer = (1 - my_y) * HALF_N

        peer_loads = []
        for k in range(C):
            cp = pltpu.make_async_copy(
                x_ref.at[0, pl.ds(row0 + k * CHUNK, CHUNK),
                         pl.ds(col_peer, HALF_N)],
                xstage.at[pl.ds(k * CHUNK, CHUNK), HALF_N:],
                peer_sems.at[k],
            )
            cp.start()
            peer_loads.append(cp)
        mine_loads = []
        for k in range(C):
            cp = pltpu.make_async_copy(
                x_ref.at[0, pl.ds(row0 + k * CHUNK, CHUNK),
                         pl.ds(col_me, HALF_N)],
                xstage.at[pl.ds(k * CHUNK, CHUNK), :HALF_N],
                mine_sems.at[k],
            )
            cp.start()
            mine_loads.append(cp)

        barrier = pltpu.get_barrier_semaphore()
        pl.semaphore_signal(
            barrier, inc=1, device_id=(my_x, 1 - my_y),
            device_id_type=pl.DeviceIdType.MESH,
        )
        pl.semaphore_signal(
            barrier, inc=1, device_id=(1 - my_x, my_y),
            device_id_type=pl.DeviceIdType.MESH,
        )
        pl.semaphore_wait(barrier, 2)

        y_rdmas = []
        for k in range(C):
            peer_loads[k].wait()
            ysend[pl.ds(k * CHUNK, CHUNK), :] = xstage[
                pl.ds(k * CHUNK, CHUNK), HALF_N:
            ].astype(jnp.bfloat16)
            rdma = pltpu.make_async_remote_copy(
                src_ref=ysend.at[pl.ds(k * CHUNK, CHUNK), :],
                dst_ref=yrecv.at[pl.ds(k * CHUNK, CHUNK), :],
                send_sem=ysend_sems.at[k],
                recv_sem=yrecv_sems.at[k],
                device_id=(my_x, 1 - my_y),
                device_id_type=pl.DeviceIdType.MESH,
            )
            rdma.start()
            y_rdmas.append(rdma)

        x_rdmas = []
        stores = []
        for k in range(C):
            mine_loads[k].wait()
            y_rdmas[k].wait_recv()
            oacc[pl.ds(k * CHUNK, CHUNK), :] = (
                xstage[pl.ds(k * CHUNK, CHUNK), :HALF_N].astype(jnp.bfloat16)
                + yrecv[pl.ds(k * CHUNK, CHUNK), :]
            )
            st = pltpu.make_async_copy(
                oacc.at[pl.ds(k * CHUNK, CHUNK), :],
                out_ref.at[pl.ds(row0 + k * CHUNK, CHUNK), :],
                store_sems.at[k],
            )
            st.start()
            stores.append(st)
            rdma = pltpu.make_async_remote_copy(
                src_ref=oacc.at[pl.ds(k * CHUNK, CHUNK), :],
                dst_ref=out_ref.at[pl.ds(row0 + k * CHUNK, CHUNK), :],
                send_sem=xsend_sems.at[k],
                recv_sem=xrecv_sems.at[k],
                device_id=(1 - my_x, my_y),
                device_id_type=pl.DeviceIdType.MESH,
            )
            rdma.start()
            x_rdmas.append(rdma)

        for k in range(C):
            stores[k].wait()
            y_rdmas[k].wait_send()
            x_rdmas[k].wait()

    res = pl.pallas_call(
        body,
        out_shape=jax.ShapeDtypeStruct((M, HALF_N), jnp.bfloat16),
        in_specs=[pl.BlockSpec(memory_space=pltpu.MemorySpace.HBM)],
        out_specs=pl.BlockSpec(memory_space=pltpu.MemorySpace.HBM),
        scratch_shapes=[
            pltpu.VMEM((HALF_M, N), jnp.float32),
            pltpu.VMEM((HALF_M, HALF_N), jnp.bfloat16),
            pltpu.VMEM((HALF_M, HALF_N), jnp.bfloat16),
            pltpu.VMEM((HALF_M, HALF_N), jnp.bfloat16),
            pltpu.SemaphoreType.DMA((C,)),
            pltpu.SemaphoreType.DMA((C,)),
            pltpu.SemaphoreType.DMA((C,)),
            pltpu.SemaphoreType.DMA((C,)),
            pltpu.SemaphoreType.DMA((C,)),
            pltpu.SemaphoreType.DMA((C,)),
            pltpu.SemaphoreType.DMA((C,)),
        ],
        compiler_params=pltpu.CompilerParams(
            collective_id=0,
            vmem_limit_bytes=120 * 1024 * 1024,
        ),
    )(x)
    return res
